# baseline (speedup 1.0000x reference)
import functools

import jax
import jax.numpy as jnp
from jax.experimental import pallas as pl

N = 10000
E = 160000
D = 128
ED = 16
H = 64
NE = 4
OUT = 40
K_LIST = (0.3, 0.5, 0.7, 0.9)

_NODE_BLK = 1000
_EDGE_BLK = 2000


def _node_precompute_body(x_ref, wsrc_ref, wdst_ref, w1_ref, wg_ref,
                          a_ref, b_ref, h0_ref, gates_ref):
    xb = x_ref[...]
    a_ref[...] = jnp.dot(xb, wsrc_ref[...], preferred_element_type=jnp.float32)
    b_ref[...] = jnp.dot(xb, wdst_ref[...], preferred_element_type=jnp.float32)
    h0_ref[...] = jnp.dot(xb, w1_ref[...], preferred_element_type=jnp.float32)
    logits = jnp.dot(xb, wg_ref[...], preferred_element_type=jnp.float32)

    def _first_occurrence(vals, m):
        cols = []
        seen = jnp.zeros_like(m)
        for j in range(NE):
            eqj = jnp.where(vals[:, j:j + 1] == m, 1.0, 0.0)
            cols.append(eqj * (1.0 - seen))
            seen = jnp.maximum(seen, eqj)
        return jnp.concatenate(cols, axis=1)

    m1 = jnp.max(logits, axis=1, keepdims=True)
    first1 = _first_occurrence(logits, m1)
    neg = jnp.where(first1 > 0.5, -jnp.inf, logits)
    m2 = jnp.max(neg, axis=1, keepdims=True)
    first2 = _first_occurrence(neg, m2)
    e2 = jnp.exp(m2 - m1)
    g1 = 1.0 / (1.0 + e2)
    g2 = 1.0 - g1
    gates4 = first1 * g1 + first2 * g2
    blk = gates_ref.shape[0]
    gates_ref[...] = jnp.concatenate(
        [gates4, jnp.zeros((blk, 16 - NE), dtype=jnp.float32)], axis=1)


def _node_precompute(x, wsrc, wdst, w1, wg):
    nblk = N // _NODE_BLK
    return pl.pallas_call(
        _node_precompute_body,
        grid=(nblk,),
        in_specs=[
            pl.BlockSpec((_NODE_BLK, D), lambda i: (i, 0)),
            pl.BlockSpec((D, NE * H), lambda i: (0, 0)),
            pl.BlockSpec((D, NE * H), lambda i: (0, 0)),
            pl.BlockSpec((D, D), lambda i: (0, 0)),
            pl.BlockSpec((D, NE), lambda i: (0, 0)),
        ],
        out_specs=[
            pl.BlockSpec((_NODE_BLK, NE * H), lambda i: (i, 0)),
            pl.BlockSpec((_NODE_BLK, NE * H), lambda i: (i, 0)),
            pl.BlockSpec((_NODE_BLK, D), lambda i: (i, 0)),
            pl.BlockSpec((_NODE_BLK, 16), lambda i: (i, 0)),
        ],
        out_shape=[
            jax.ShapeDtypeStruct((N, NE * H), jnp.float32),
            jax.ShapeDtypeStruct((N, NE * H), jnp.float32),
            jax.ShapeDtypeStruct((N, D), jnp.float32),
            jax.ShapeDtypeStruct((N, 16), jnp.float32),
        ],
    )(x, wsrc, wdst, w1, wg)


def _edge_c_body(ea_ref, wattr_ref, be1_ref, c_ref):
    c_ref[...] = (jnp.dot(ea_ref[...], wattr_ref[...],
                          preferred_element_type=jnp.float32)
                  + be1_ref[...])


def _edge_c(edge_attr, wattr, be1flat):
    nblk = E // _EDGE_BLK
    return pl.pallas_call(
        _edge_c_body,
        grid=(nblk,),
        in_specs=[
            pl.BlockSpec((_EDGE_BLK, ED), lambda i: (i, 0)),
            pl.BlockSpec((ED, NE * H), lambda i: (0, 0)),
            pl.BlockSpec((1, NE * H), lambda i: (0, 0)),
        ],
        out_specs=pl.BlockSpec((_EDGE_BLK, NE * H), lambda i: (i, 0)),
        out_shape=jax.ShapeDtypeStruct((E, NE * H), jnp.float32),
    )(edge_attr, wattr, be1flat.reshape(1, NE * H))


def _edge_combine_body(as_ref, bd_ref, c_ref, gd_ref, w2_ref, be2_ref, ew_ref):
    blk = as_ref.shape[0]
    pre = as_ref[...] + bd_ref[...] + c_ref[...]
    hrelu = jnp.maximum(pre, 0.0)
    w2 = w2_ref[...]
    prod = (hrelu * w2).reshape(blk, NE, H)
    s = jnp.sum(prod, axis=-1) + be2_ref[...]
    sig = 1.0 / (1.0 + jnp.exp(-s))
    gd = gd_ref[...]
    ew = sig[:, 0] * K_LIST[0] * gd[:, 0]
    for n in range(1, NE):
        ew = ew + sig[:, n] * K_LIST[n] * gd[:, n]
    ew_ref[...] = ew.reshape(1, 1, blk)


def _edge_combine(a_src, b_dst, c, gates_dst, w2flat, be2row):
    nblk = E // _EDGE_BLK
    ew = pl.pallas_call(
        _edge_combine_body,
        grid=(nblk,),
        in_specs=[
            pl.BlockSpec((_EDGE_BLK, NE * H), lambda i: (i, 0)),
            pl.BlockSpec((_EDGE_BLK, NE * H), lambda i: (i, 0)),
            pl.BlockSpec((_EDGE_BLK, NE * H), lambda i: (i, 0)),
            pl.BlockSpec((_EDGE_BLK, 16), lambda i: (i, 0)),
            pl.BlockSpec((1, NE * H), lambda i: (0, 0)),
            pl.BlockSpec((1, NE), lambda i: (0, 0)),
        ],
        out_specs=pl.BlockSpec((1, 1, _EDGE_BLK), lambda i: (i, 0, 0)),
        out_shape=jax.ShapeDtypeStruct((nblk, 1, _EDGE_BLK), jnp.float32),
    )(a_src, b_dst, c, gates_dst, w2flat, be2row)
    return ew.reshape(E)


def kernel(x, edge_index, edge_attr, Wg, We1, be1, We2, be2, W1, W2):
    src = edge_index[0]
    dst = edge_index[1]
    wflat = We1.transpose(1, 0, 2).reshape(2 * D + ED, NE * H)
    wsrc = wflat[:D]
    wdst = wflat[D:2 * D]
    wattr = wflat[2 * D:]
    be1flat = be1.reshape(NE * H)
    w2flat = We2[:, :, 0].reshape(1, NE * H)
    be2row = be2.reshape(1, NE)

    a, b, h0, gates = _node_precompute(x, wsrc, wdst, W1, Wg)
    c = _edge_c(edge_attr, wattr, be1flat)

    ew = _edge_combine(a[src], b[dst], c, gates[dst], w2flat, be2row)

    deg = jax.ops.segment_sum(ew, dst, num_segments=N) + 1.0
    inv = jax.lax.rsqrt(deg)
    nw = ew * inv[src] * inv[dst]
    h1 = jax.nn.relu(jax.ops.segment_sum(h0[src] * nw[:, None], dst,
                                         num_segments=N))
    h2 = h1 @ W2
    out = jax.ops.segment_sum(h2[src] * nw[:, None], dst, num_segments=N)
    return out


# speedup vs baseline: 1.4227x; 1.4227x over previous
import functools

import jax
import jax.numpy as jnp
from jax import lax
from jax.experimental import pallas as pl
from jax.experimental.pallas import tpu as pltpu
from jax.experimental.pallas import tpu_sc as plsc

N = 10000
E = 160000
D = 128
ED = 16
H = 64
NE = 4
OUT = 40
K_LIST = (0.3, 0.5, 0.7, 0.9)

_NODE_BLK = 1000
_EDGE_BLK = 2000


def _node_precompute_body(x_ref, wsrc_ref, wdst_ref, w1_ref, wg_ref,
                          a_ref, b_ref, h0_ref, gates_ref):
    xb = x_ref[...]
    a_ref[...] = jnp.dot(xb, wsrc_ref[...], preferred_element_type=jnp.float32)
    b_ref[...] = jnp.dot(xb, wdst_ref[...], preferred_element_type=jnp.float32)
    h0_ref[...] = jnp.dot(xb, w1_ref[...], preferred_element_type=jnp.float32)
    logits = jnp.dot(xb, wg_ref[...], preferred_element_type=jnp.float32)

    def _first_occurrence(vals, m):
        cols = []
        seen = jnp.zeros_like(m)
        for j in range(NE):
            eqj = jnp.where(vals[:, j:j + 1] == m, 1.0, 0.0)
            cols.append(eqj * (1.0 - seen))
            seen = jnp.maximum(seen, eqj)
        return jnp.concatenate(cols, axis=1)

    m1 = jnp.max(logits, axis=1, keepdims=True)
    first1 = _first_occurrence(logits, m1)
    neg = jnp.where(first1 > 0.5, -jnp.inf, logits)
    m2 = jnp.max(neg, axis=1, keepdims=True)
    first2 = _first_occurrence(neg, m2)
    e2 = jnp.exp(m2 - m1)
    g1 = 1.0 / (1.0 + e2)
    g2 = 1.0 - g1
    gates_ref[...] = first1 * g1 + first2 * g2


def _node_precompute(x, wsrc, wdst, w1, wg):
    nblk = N // _NODE_BLK
    return pl.pallas_call(
        _node_precompute_body,
        grid=(nblk,),
        in_specs=[
            pl.BlockSpec((_NODE_BLK, D), lambda i: (i, 0)),
            pl.BlockSpec((D, NE * H), lambda i: (0, 0)),
            pl.BlockSpec((D, NE * H), lambda i: (0, 0)),
            pl.BlockSpec((D, D), lambda i: (0, 0)),
            pl.BlockSpec((D, NE), lambda i: (0, 0)),
        ],
        out_specs=[
            pl.BlockSpec((_NODE_BLK, NE * H), lambda i: (i, 0)),
            pl.BlockSpec((_NODE_BLK, NE * H), lambda i: (i, 0)),
            pl.BlockSpec((_NODE_BLK, D), lambda i: (i, 0)),
            pl.BlockSpec((_NODE_BLK, NE), lambda i: (i, 0)),
        ],
        out_shape=[
            jax.ShapeDtypeStruct((N, NE * H), jnp.float32),
            jax.ShapeDtypeStruct((N, NE * H), jnp.float32),
            jax.ShapeDtypeStruct((N, D), jnp.float32),
            jax.ShapeDtypeStruct((N, NE), jnp.float32),
        ],
    )(x, wsrc, wdst, w1, wg)


def _edge_c_body(ea_ref, wattr_ref, be1_ref, c_ref):
    c_ref[...] = (jnp.dot(ea_ref[...], wattr_ref[...],
                          preferred_element_type=jnp.float32)
                  + be1_ref[...])


def _edge_c(edge_attr, wattr, be1flat):
    nblk = E // _EDGE_BLK
    return pl.pallas_call(
        _edge_c_body,
        grid=(nblk,),
        in_specs=[
            pl.BlockSpec((_EDGE_BLK, ED), lambda i: (i, 0)),
            pl.BlockSpec((ED, NE * H), lambda i: (0, 0)),
            pl.BlockSpec((1, NE * H), lambda i: (0, 0)),
        ],
        out_specs=pl.BlockSpec((_EDGE_BLK, NE * H), lambda i: (i, 0)),
        out_shape=jax.ShapeDtypeStruct((E, NE * H), jnp.float32),
    )(edge_attr, wattr, be1flat.reshape(1, NE * H))


def _edge_combine_body(as_ref, bd_ref, c_ref, gd_ref, w2_ref, be2_ref, ew_ref):
    blk = as_ref.shape[0]
    pre = as_ref[...] + bd_ref[...] + c_ref[...]
    hrelu = jnp.maximum(pre, 0.0)
    w2 = w2_ref[...]
    prod = (hrelu * w2).reshape(blk, NE, H)
    s = jnp.sum(prod, axis=-1) + be2_ref[...]
    sig = 1.0 / (1.0 + jnp.exp(-s))
    gd = gd_ref[...]
    ew = sig[:, 0] * K_LIST[0] * gd[:, 0]
    for n in range(1, NE):
        ew = ew + sig[:, n] * K_LIST[n] * gd[:, n]
    ew_ref[...] = ew.reshape(1, 1, blk)


def _edge_combine(a_src, b_dst, c, gates_dst, w2flat, be2row):
    nblk = E // _EDGE_BLK
    ew = pl.pallas_call(
        _edge_combine_body,
        grid=(nblk,),
        in_specs=[
            pl.BlockSpec((_EDGE_BLK, NE * H), lambda i: (i, 0)),
            pl.BlockSpec((_EDGE_BLK, NE * H), lambda i: (i, 0)),
            pl.BlockSpec((_EDGE_BLK, NE * H), lambda i: (i, 0)),
            pl.BlockSpec((_EDGE_BLK, 16), lambda i: (i, 0)),
            pl.BlockSpec((1, NE * H), lambda i: (0, 0)),
            pl.BlockSpec((1, NE), lambda i: (0, 0)),
        ],
        out_specs=pl.BlockSpec((1, 1, _EDGE_BLK), lambda i: (i, 0, 0)),
        out_shape=jax.ShapeDtypeStruct((nblk, 1, _EDGE_BLK), jnp.float32),
    )(a_src, b_dst, c, gates_dst, w2flat, be2row)
    return ew.reshape(E)


_NC = 2
_NS = 16
_NW = _NC * _NS
_CH = 80
_NCHUNK = E // _CH
_FW = NE * H


def _sc_pass1_body(a_hbm, b_hbm, c_hbm, g_hbm, src_hbm, dst_hbm, w2_hbm,
                   be2_hbm,
                   ew_hbm, deg_hbm,
                   srcb, dstb, bufa, bufb, bufc, sbuf, ewbuf,
                   w2v, be2v, gatesv, zbuf, deg_sh):
    core = lax.axis_index("c")
    sub = lax.axis_index("s")
    wid = sub * _NC + core

    for i in range(63):
        zbuf[pl.ds(16 * i, 16)] = jnp.zeros((16,), jnp.float32)

    @pl.when(sub < 10)
    def _():
        pltpu.sync_copy(zbuf.at[pl.ds(0, 1000)],
                        deg_sh.at[pl.ds(sub * 1000, 1000)])
    pltpu.sync_copy(w2_hbm, w2v)
    pltpu.sync_copy(be2_hbm, be2v)
    pltpu.sync_copy(g_hbm, gatesv)
    plsc.subcore_barrier()

    base_nk = _NCHUNK // _NW
    nk = jnp.where(wid < _NCHUNK % _NW, base_nk + 1, base_nk)

    def chunk_body(k, _):
        ci = wid + _NW * k
        e0 = ci * _CH
        pltpu.sync_copy(src_hbm.at[pl.ds(e0, _CH)], srcb)
        pltpu.sync_copy(dst_hbm.at[pl.ds(e0, _CH)], dstb)
        pltpu.sync_copy(a_hbm.at[srcb], bufa)
        pltpu.sync_copy(b_hbm.at[dstb], bufb)
        pltpu.sync_copy(c_hbm.at[pl.ds(e0, _CH)], bufc)

        @plsc.parallel_loop(0, _CH, 1, unroll=2)
        def edge_body(e):
            accs = []
            for n in range(NE):
                acc = jnp.zeros((16,), jnp.float32)
                for jj in range(H // 16):
                    j = n * (H // 16) + jj
                    va = bufa[e, pl.ds(16 * j, 16)]
                    vb = bufb[e, pl.ds(16 * j, 16)]
                    vc = bufc[e, pl.ds(16 * j, 16)]
                    r = jnp.maximum(va + vb + vc, 0.0)
                    acc = acc + r * w2v[pl.ds(16 * j, 16)]
                accs.append(acc)
            for n in range(NE):
                sbuf[pl.ds(e * 64 + n * 16, 16)] = accs[n]

        for g in range(_CH // 16):
            lanebase = lax.iota(jnp.int32, 16) + g * 16
            dstv = dstb[pl.ds(g * 16, 16)]
            ewv = jnp.zeros((16,), jnp.float32)
            for n in range(NE):
                sv = jnp.zeros((16,), jnp.float32)
                for t in range(16):
                    sv = sv + plsc.load_gather(
                        sbuf, [lanebase * 64 + (n * 16 + t)])
                sv = sv + plsc.load_gather(
                    be2v, [jnp.full((16,), n, jnp.int32)])
                sig = 1.0 / (1.0 + jnp.exp(-sv))
                gn = plsc.load_gather(gatesv, [dstv * NE + n])
                ewv = ewv + sig * K_LIST[n] * gn
            ewbuf[pl.ds(g * 16, 16)] = ewv

        pltpu.sync_copy(ewbuf, ew_hbm.at[pl.ds(e0, _CH)])
        pltpu.sync_copy(ewbuf, deg_sh.at[dstb], add=True)
        return ()

    lax.fori_loop(0, nk, chunk_body, ())

    plsc.subcore_barrier()
    @pl.when(sub < 10)
    def _():
        pltpu.sync_copy(deg_sh.at[pl.ds(sub * 1000, 1000)],
                        zbuf.at[pl.ds(0, 1000)])
        pltpu.sync_copy(zbuf.at[pl.ds(0, 1000)],
                        deg_hbm.at[pl.ds(core * N + sub * 1000, 1000)])


def _sc_pass1(a, b, c, gates, src, dst, w2flat1d, be2pad):
    mesh = plsc.VectorSubcoreMesh(core_axis_name="c", subcore_axis_name="s")
    f = pl.kernel(
        _sc_pass1_body,
        out_type=[
            jax.ShapeDtypeStruct((E,), jnp.float32),
            jax.ShapeDtypeStruct((_NC * N,), jnp.float32),
        ],
        mesh=mesh,
        compiler_params=pltpu.CompilerParams(needs_layout_passes=False),
        scratch_types=[
            pltpu.VMEM((_CH,), jnp.int32),
            pltpu.VMEM((_CH,), jnp.int32),
            pltpu.VMEM((_CH, _FW), jnp.float32),
            pltpu.VMEM((_CH, _FW), jnp.float32),
            pltpu.VMEM((_CH, _FW), jnp.float32),
            pltpu.VMEM((_CH * 64,), jnp.float32),
            pltpu.VMEM((_CH,), jnp.float32),
            pltpu.VMEM((_FW,), jnp.float32),
            pltpu.VMEM((16,), jnp.float32),
            pltpu.VMEM((N * NE,), jnp.float32),
            pltpu.VMEM((1008,), jnp.float32),
            pltpu.VMEM_SHARED((N,), jnp.float32),
        ],
    )
    return f(a, b, c, gates, src, dst, w2flat1d, be2pad)


def kernel(x, edge_index, edge_attr, Wg, We1, be1, We2, be2, W1, W2):
    src = edge_index[0]
    dst = edge_index[1]
    wflat = We1.transpose(1, 0, 2).reshape(2 * D + ED, NE * H)
    wsrc = wflat[:D]
    wdst = wflat[D:2 * D]
    wattr = wflat[2 * D:]
    be1flat = be1.reshape(NE * H)
    w2flat = We2[:, :, 0].reshape(1, NE * H)
    be2row = be2.reshape(1, NE)

    a, b, h0, gates = _node_precompute(x, wsrc, wdst, W1, Wg)
    c = _edge_c(edge_attr, wattr, be1flat)

    be2pad = jnp.concatenate([be2.reshape(NE),
                              jnp.zeros((16 - NE,), jnp.float32)])
    ew, deg_parts = _sc_pass1(a, b, c, gates.reshape(N * NE), src, dst,
                              w2flat.reshape(NE * H), be2pad)

    deg = deg_parts[:N] + deg_parts[N:] + 1.0
    inv = jax.lax.rsqrt(deg)
    nw = ew * inv[src] * inv[dst]
    h1 = jax.nn.relu(jax.ops.segment_sum(h0[src] * nw[:, None], dst,
                                         num_segments=N))
    h2 = h1 @ W2
    out = jax.ops.segment_sum(h2[src] * nw[:, None], dst, num_segments=N)
    return out


# speedup vs baseline: 4.1724x; 2.9327x over previous
import functools

import jax
import jax.numpy as jnp
from jax import lax
from jax.experimental import pallas as pl
from jax.experimental.pallas import tpu as pltpu
from jax.experimental.pallas import tpu_sc as plsc

N = 10000
E = 160000
D = 128
ED = 16
H = 64
NE = 4
OUT = 40
K_LIST = (0.3, 0.5, 0.7, 0.9)

_NODE_BLK = 1000
_EDGE_BLK = 2000


def _node_precompute_body(x_ref, wsrc_ref, wdst_ref, w1_ref, wg_ref,
                          a_ref, b_ref, h0_ref, gates_ref):
    xb = x_ref[...]
    a_ref[...] = jnp.dot(xb, wsrc_ref[...], preferred_element_type=jnp.float32)
    b_ref[...] = jnp.dot(xb, wdst_ref[...], preferred_element_type=jnp.float32)
    h0_ref[...] = jnp.dot(xb, w1_ref[...], preferred_element_type=jnp.float32)
    logits = jnp.dot(xb, wg_ref[...], preferred_element_type=jnp.float32)

    def _first_occurrence(vals, m):
        cols = []
        seen = jnp.zeros_like(m)
        for j in range(NE):
            eqj = jnp.where(vals[:, j:j + 1] == m, 1.0, 0.0)
            cols.append(eqj * (1.0 - seen))
            seen = jnp.maximum(seen, eqj)
        return jnp.concatenate(cols, axis=1)

    m1 = jnp.max(logits, axis=1, keepdims=True)
    first1 = _first_occurrence(logits, m1)
    neg = jnp.where(first1 > 0.5, -jnp.inf, logits)
    m2 = jnp.max(neg, axis=1, keepdims=True)
    first2 = _first_occurrence(neg, m2)
    e2 = jnp.exp(m2 - m1)
    g1 = 1.0 / (1.0 + e2)
    g2 = 1.0 - g1
    gates_ref[...] = first1 * g1 + first2 * g2


def _node_precompute(x, wsrc, wdst, w1, wg):
    nblk = N // _NODE_BLK
    return pl.pallas_call(
        _node_precompute_body,
        grid=(nblk,),
        in_specs=[
            pl.BlockSpec((_NODE_BLK, D), lambda i: (i, 0)),
            pl.BlockSpec((D, NE * H), lambda i: (0, 0)),
            pl.BlockSpec((D, NE * H), lambda i: (0, 0)),
            pl.BlockSpec((D, D), lambda i: (0, 0)),
            pl.BlockSpec((D, NE), lambda i: (0, 0)),
        ],
        out_specs=[
            pl.BlockSpec((_NODE_BLK, NE * H), lambda i: (i, 0)),
            pl.BlockSpec((_NODE_BLK, NE * H), lambda i: (i, 0)),
            pl.BlockSpec((_NODE_BLK, D), lambda i: (i, 0)),
            pl.BlockSpec((_NODE_BLK, NE), lambda i: (i, 0)),
        ],
        out_shape=[
            jax.ShapeDtypeStruct((N, NE * H), jnp.float32),
            jax.ShapeDtypeStruct((N, NE * H), jnp.float32),
            jax.ShapeDtypeStruct((N, D), jnp.float32),
            jax.ShapeDtypeStruct((N, NE), jnp.float32),
        ],
    )(x, wsrc, wdst, w1, wg)


def _edge_c_body(ea_ref, wattr_ref, be1_ref, c_ref):
    c_ref[...] = (jnp.dot(ea_ref[...], wattr_ref[...],
                          preferred_element_type=jnp.float32)
                  + be1_ref[...])


def _edge_c(edge_attr, wattr, be1flat):
    nblk = E // _EDGE_BLK
    return pl.pallas_call(
        _edge_c_body,
        grid=(nblk,),
        in_specs=[
            pl.BlockSpec((_EDGE_BLK, ED), lambda i: (i, 0)),
            pl.BlockSpec((ED, NE * H), lambda i: (0, 0)),
            pl.BlockSpec((1, NE * H), lambda i: (0, 0)),
        ],
        out_specs=pl.BlockSpec((_EDGE_BLK, NE * H), lambda i: (i, 0)),
        out_shape=jax.ShapeDtypeStruct((E, NE * H), jnp.float32),
    )(edge_attr, wattr, be1flat.reshape(1, NE * H))


def _edge_combine_body(as_ref, bd_ref, c_ref, gd_ref, w2_ref, be2_ref, ew_ref):
    blk = as_ref.shape[0]
    pre = as_ref[...] + bd_ref[...] + c_ref[...]
    hrelu = jnp.maximum(pre, 0.0)
    w2 = w2_ref[...]
    prod = (hrelu * w2).reshape(blk, NE, H)
    s = jnp.sum(prod, axis=-1) + be2_ref[...]
    sig = 1.0 / (1.0 + jnp.exp(-s))
    gd = gd_ref[...]
    ew = sig[:, 0] * K_LIST[0] * gd[:, 0]
    for n in range(1, NE):
        ew = ew + sig[:, n] * K_LIST[n] * gd[:, n]
    ew_ref[...] = ew.reshape(1, 1, blk)


def _edge_combine(a_src, b_dst, c, gates_dst, w2flat, be2row):
    nblk = E // _EDGE_BLK
    ew = pl.pallas_call(
        _edge_combine_body,
        grid=(nblk,),
        in_specs=[
            pl.BlockSpec((_EDGE_BLK, NE * H), lambda i: (i, 0)),
            pl.BlockSpec((_EDGE_BLK, NE * H), lambda i: (i, 0)),
            pl.BlockSpec((_EDGE_BLK, NE * H), lambda i: (i, 0)),
            pl.BlockSpec((_EDGE_BLK, 16), lambda i: (i, 0)),
            pl.BlockSpec((1, NE * H), lambda i: (0, 0)),
            pl.BlockSpec((1, NE), lambda i: (0, 0)),
        ],
        out_specs=pl.BlockSpec((1, 1, _EDGE_BLK), lambda i: (i, 0, 0)),
        out_shape=jax.ShapeDtypeStruct((nblk, 1, _EDGE_BLK), jnp.float32),
    )(a_src, b_dst, c, gates_dst, w2flat, be2row)
    return ew.reshape(E)


_NC = 2
_NS = 16
_NW = _NC * _NS
_CH = 80
_NCHUNK = E // _CH
_FW = NE * H


def _sc_pass1_body(a_hbm, b_hbm, c_hbm, g_hbm, src_hbm, dst_hbm, w2_hbm,
                   be2_hbm,
                   ew_hbm, deg_hbm,
                   srcb, dstb, bufa, bufb, bufc, sbuf, ewbuf,
                   w2v, be2v, gatesv, zbuf, deg_sh):
    core = lax.axis_index("c")
    sub = lax.axis_index("s")
    wid = sub * _NC + core

    for i in range(63):
        zbuf[pl.ds(16 * i, 16)] = jnp.zeros((16,), jnp.float32)

    @pl.when(sub < 10)
    def _():
        pltpu.sync_copy(zbuf.at[pl.ds(0, 1000)],
                        deg_sh.at[pl.ds(sub * 1000, 1000)])
    pltpu.sync_copy(w2_hbm, w2v)
    pltpu.sync_copy(be2_hbm, be2v)
    pltpu.sync_copy(g_hbm, gatesv)
    plsc.subcore_barrier()

    base_nk = _NCHUNK // _NW
    nk = jnp.where(wid < _NCHUNK % _NW, base_nk + 1, base_nk)

    def chunk_body(k, _):
        ci = wid + _NW * k
        e0 = ci * _CH
        pltpu.sync_copy(src_hbm.at[pl.ds(e0, _CH)], srcb)
        pltpu.sync_copy(dst_hbm.at[pl.ds(e0, _CH)], dstb)
        pltpu.sync_copy(a_hbm.at[srcb], bufa)
        pltpu.sync_copy(b_hbm.at[dstb], bufb)
        pltpu.sync_copy(c_hbm.at[pl.ds(e0, _CH)], bufc)

        @plsc.parallel_loop(0, _CH, 1, unroll=2)
        def edge_body(e):
            accs = []
            for n in range(NE):
                acc = jnp.zeros((16,), jnp.float32)
                for jj in range(H // 16):
                    j = n * (H // 16) + jj
                    va = bufa[e, pl.ds(16 * j, 16)]
                    vb = bufb[e, pl.ds(16 * j, 16)]
                    vc = bufc[e, pl.ds(16 * j, 16)]
                    r = jnp.maximum(va + vb + vc, 0.0)
                    acc = acc + r * w2v[pl.ds(16 * j, 16)]
                accs.append(acc)
            for n in range(NE):
                sbuf[pl.ds(e * 64 + n * 16, 16)] = accs[n]

        for g in range(_CH // 16):
            lanebase = lax.iota(jnp.int32, 16) + g * 16
            dstv = dstb[pl.ds(g * 16, 16)]
            ewv = jnp.zeros((16,), jnp.float32)
            for n in range(NE):
                sv = jnp.zeros((16,), jnp.float32)
                for t in range(16):
                    sv = sv + plsc.load_gather(
                        sbuf, [lanebase * 64 + (n * 16 + t)])
                sv = sv + plsc.load_gather(
                    be2v, [jnp.full((16,), n, jnp.int32)])
                sig = 1.0 / (1.0 + jnp.exp(-sv))
                gn = plsc.load_gather(gatesv, [dstv * NE + n])
                ewv = ewv + sig * K_LIST[n] * gn
            ewbuf[pl.ds(g * 16, 16)] = ewv

        pltpu.sync_copy(ewbuf, ew_hbm.at[pl.ds(e0, _CH)])
        pltpu.sync_copy(ewbuf, deg_sh.at[dstb], add=True)
        return ()

    lax.fori_loop(0, nk, chunk_body, ())

    plsc.subcore_barrier()
    @pl.when(sub < 10)
    def _():
        pltpu.sync_copy(deg_sh.at[pl.ds(sub * 1000, 1000)],
                        zbuf.at[pl.ds(0, 1000)])
        pltpu.sync_copy(zbuf.at[pl.ds(0, 1000)],
                        deg_hbm.at[pl.ds(core * N + sub * 1000, 1000)])


def _sc_pass1(a, b, c, gates, src, dst, w2flat1d, be2pad):
    mesh = plsc.VectorSubcoreMesh(core_axis_name="c", subcore_axis_name="s")
    f = pl.kernel(
        _sc_pass1_body,
        out_type=[
            jax.ShapeDtypeStruct((E,), jnp.float32),
            jax.ShapeDtypeStruct((_NC * N,), jnp.float32),
        ],
        mesh=mesh,
        compiler_params=pltpu.CompilerParams(needs_layout_passes=False),
        scratch_types=[
            pltpu.VMEM((_CH,), jnp.int32),
            pltpu.VMEM((_CH,), jnp.int32),
            pltpu.VMEM((_CH, _FW), jnp.float32),
            pltpu.VMEM((_CH, _FW), jnp.float32),
            pltpu.VMEM((_CH, _FW), jnp.float32),
            pltpu.VMEM((_CH * 64,), jnp.float32),
            pltpu.VMEM((_CH,), jnp.float32),
            pltpu.VMEM((_FW,), jnp.float32),
            pltpu.VMEM((16,), jnp.float32),
            pltpu.VMEM((N * NE,), jnp.float32),
            pltpu.VMEM((1008,), jnp.float32),
            pltpu.VMEM_SHARED((N,), jnp.float32),
        ],
    )
    return f(a, b, c, gates, src, dst, w2flat1d, be2pad)


_ZROWS = 200


def _make_sc_scatter_body(width, with_nw):
    def body(*refs):
        if with_nw:
            (h_hbm, ew_hbm, src_hbm, dst_hbm, inv_hbm, z_hbm,
             nw_hbm, acc_hbm,
             srcb, dstb, ewb, nwb, bufh, vstage, inv_v, acc_sh) = refs
        else:
            (h_hbm, nw_hbm_in, src_hbm, dst_hbm, z_hbm,
             acc_hbm,
             srcb, dstb, nwb, bufh, vstage, acc_sh) = refs
        core = lax.axis_index("c")
        sub = lax.axis_index("s")
        wid = sub * _NC + core

        pltpu.sync_copy(z_hbm, vstage)

        @pl.when(sub < 10)
        def _():
            for i in range(1000 // _ZROWS):
                pltpu.sync_copy(
                    vstage.at[...],
                    acc_sh.at[pl.ds(sub * 1000 + i * _ZROWS, _ZROWS), :])
        if with_nw:
            pltpu.sync_copy(inv_hbm, inv_v)
        plsc.subcore_barrier()

        nchunk = E // _CH
        base_nk = nchunk // _NW
        nk = jnp.where(wid < nchunk % _NW, base_nk + 1, base_nk)

        def chunk_body(k, _):
            ci = wid + _NW * k
            e0 = ci * _CH
            pltpu.sync_copy(src_hbm.at[pl.ds(e0, _CH)], srcb)
            pltpu.sync_copy(dst_hbm.at[pl.ds(e0, _CH)], dstb)
            if with_nw:
                pltpu.sync_copy(ew_hbm.at[pl.ds(e0, _CH)], ewb)
                for g in range(_CH // 16):
                    vsrc = srcb[pl.ds(g * 16, 16)]
                    vdst = dstb[pl.ds(g * 16, 16)]
                    invs = plsc.load_gather(inv_v, [vsrc])
                    invd = plsc.load_gather(inv_v, [vdst])
                    nwb[pl.ds(g * 16, 16)] = (
                        ewb[pl.ds(g * 16, 16)] * invs * invd)
                pltpu.sync_copy(nwb, nw_hbm.at[pl.ds(e0, _CH)])
            else:
                pltpu.sync_copy(nw_hbm_in.at[pl.ds(e0, _CH)], nwb)

            pltpu.sync_copy(h_hbm.at[srcb], bufh)

            @plsc.parallel_loop(0, _CH, 1, unroll=2)
            def edge_body(e):
                nws = plsc.load_gather(nwb, [jnp.full((16,), e, jnp.int32)])
                for j in range(width // 16):
                    bufh[e, pl.ds(16 * j, 16)] = (
                        bufh[e, pl.ds(16 * j, 16)] * nws)

            pltpu.sync_copy(bufh, acc_sh.at[dstb], add=True)
            return ()

        lax.fori_loop(0, nk, chunk_body, ())

        plsc.subcore_barrier()

        @pl.when(sub < 10)
        def _():
            for i in range(1000 // _ZROWS):
                r = sub * 1000 + i * _ZROWS
                pltpu.sync_copy(acc_sh.at[pl.ds(r, _ZROWS), :], vstage)
                pltpu.sync_copy(
                    vstage.at[...],
                    acc_hbm.at[pl.ds(core * N + r, _ZROWS), :])

    return body


def _sc_pass2(h0, ew, src, dst, inv):
    mesh = plsc.VectorSubcoreMesh(core_axis_name="c", subcore_axis_name="s")
    f = pl.kernel(
        _make_sc_scatter_body(D, True),
        out_type=[
            jax.ShapeDtypeStruct((E,), jnp.float32),
            jax.ShapeDtypeStruct((_NC * N, D), jnp.float32),
        ],
        mesh=mesh,
        compiler_params=pltpu.CompilerParams(needs_layout_passes=False),
        scratch_types=[
            pltpu.VMEM((_CH,), jnp.int32),
            pltpu.VMEM((_CH,), jnp.int32),
            pltpu.VMEM((_CH,), jnp.float32),
            pltpu.VMEM((_CH,), jnp.float32),
            pltpu.VMEM((_CH, D), jnp.float32),
            pltpu.VMEM((_ZROWS, D), jnp.float32),
            pltpu.VMEM((N,), jnp.float32),
            pltpu.VMEM_SHARED((N, D), jnp.float32),
        ],
    )
    return f(h0, ew, src, dst, inv, jnp.zeros((_ZROWS, D), jnp.float32))


def _sc_pass3(h2p, nw, src, dst):
    mesh = plsc.VectorSubcoreMesh(core_axis_name="c", subcore_axis_name="s")
    f = pl.kernel(
        _make_sc_scatter_body(D, False),
        out_type=[
            jax.ShapeDtypeStruct((_NC * N, D), jnp.float32),
        ],
        mesh=mesh,
        compiler_params=pltpu.CompilerParams(needs_layout_passes=False),
        scratch_types=[
            pltpu.VMEM((_CH,), jnp.int32),
            pltpu.VMEM((_CH,), jnp.int32),
            pltpu.VMEM((_CH,), jnp.float32),
            pltpu.VMEM((_CH, D), jnp.float32),
            pltpu.VMEM((_ZROWS, D), jnp.float32),
            pltpu.VMEM_SHARED((N, D), jnp.float32),
        ],
    )
    return f(h2p, nw, src, dst, jnp.zeros((_ZROWS, D), jnp.float32))


def _inv_body(dp_ref, inv_ref):
    inv_ref[...] = lax.rsqrt(dp_ref[0] + dp_ref[1] + 1.0)[None]


def _tc_inv(deg_parts2):
    return pl.pallas_call(
        _inv_body,
        out_shape=jax.ShapeDtypeStruct((1, N), jnp.float32),
    )(deg_parts2).reshape(N)


def _h2_body(p0_ref, p1_ref, w2_ref, h2_ref):
    hb = jnp.maximum(p0_ref[...] + p1_ref[...], 0.0)
    h2_ref[...] = jnp.dot(hb, w2_ref[...], preferred_element_type=jnp.float32)


def _tc_h2(h1a, h1b, w2pad):
    nblk = N // _NODE_BLK
    return pl.pallas_call(
        _h2_body,
        grid=(nblk,),
        in_specs=[
            pl.BlockSpec((_NODE_BLK, D), lambda i: (i, 0)),
            pl.BlockSpec((_NODE_BLK, D), lambda i: (i, 0)),
            pl.BlockSpec((D, D), lambda i: (0, 0)),
        ],
        out_specs=pl.BlockSpec((_NODE_BLK, D), lambda i: (i, 0)),
        out_shape=jax.ShapeDtypeStruct((N, D), jnp.float32),
    )(h1a, h1b, w2pad)


def _final_body(p0_ref, p1_ref, out_ref):
    out_ref[...] = (p0_ref[...] + p1_ref[...])[:, :OUT]


def _tc_final(p0, p1):
    nblk = N // _NODE_BLK
    return pl.pallas_call(
        _final_body,
        grid=(nblk,),
        in_specs=[
            pl.BlockSpec((_NODE_BLK, D), lambda i: (i, 0)),
            pl.BlockSpec((_NODE_BLK, D), lambda i: (i, 0)),
        ],
        out_specs=pl.BlockSpec((_NODE_BLK, OUT), lambda i: (i, 0)),
        out_shape=jax.ShapeDtypeStruct((N, OUT), jnp.float32),
    )(p0, p1)


def kernel(x, edge_index, edge_attr, Wg, We1, be1, We2, be2, W1, W2):
    src = edge_index[0]
    dst = edge_index[1]
    wflat = We1.transpose(1, 0, 2).reshape(2 * D + ED, NE * H)
    wsrc = wflat[:D]
    wdst = wflat[D:2 * D]
    wattr = wflat[2 * D:]
    be1flat = be1.reshape(NE * H)
    w2flat = We2[:, :, 0].reshape(1, NE * H)
    be2row = be2.reshape(1, NE)

    a, b, h0, gates = _node_precompute(x, wsrc, wdst, W1, Wg)
    c = _edge_c(edge_attr, wattr, be1flat)

    be2pad = jnp.concatenate([be2.reshape(NE),
                              jnp.zeros((16 - NE,), jnp.float32)])
    ew, deg_parts = _sc_pass1(a, b, c, gates.reshape(N * NE), src, dst,
                              w2flat.reshape(NE * H), be2pad)

    inv = _tc_inv(deg_parts.reshape(2, N))
    nw, h1_parts = _sc_pass2(h0, ew, src, dst, inv)
    w2pad = jnp.concatenate(
        [W2, jnp.zeros((D, D - OUT), jnp.float32)], axis=1)
    h2p = _tc_h2(h1_parts[:N], h1_parts[N:], w2pad)
    out_parts, = _sc_pass3(h2p, nw, src, dst)
    return _tc_final(out_parts[:N], out_parts[N:])
